# baseline (speedup 1.0000x reference)
import functools

import jax
import jax.numpy as jnp
from jax import lax
from jax.experimental import pallas as pl
from jax.experimental.pallas import tpu as pltpu
from jax.experimental.pallas import tpu_sc as plsc

_B, _L, _D = 4, 8192, 768
_LANES = 16
_SUB = 16
_NC, _NS = 2, 16
_NW = _NC * _NS
_LW = _L // _NW
_NSUB = _LW // _SUB
_DV = _D // _LANES


def _sc_body(emb_hbm, sym_hbm, wpos_hbm, out_hbm, emb_v, wpos_v, sym_v):
    c = lax.axis_index("c")
    s = lax.axis_index("s")
    wid = s * _NC + c
    l0w = wid * _LW

    def sub_body(k, carry):
        l0 = l0w + k * _SUB
        pltpu.sync_copy(wpos_hbm.at[pl.ds(l0, _SUB)], wpos_v)

        def b_body(b, carry2):
            base = b * _L + l0
            pltpu.sync_copy(emb_hbm.at[pl.ds(base, _SUB)], emb_v)
            pltpu.sync_copy(sym_hbm.at[pl.ds(base, _SUB)], sym_v)

            svec = sym_v[pl.ds(0, _SUB)]
            mvec = jnp.where(svec != 0, jnp.float32(1.0), jnp.float32(0.0))
            for r in range(_SUB):
                m = mvec[r]
                for j in range(_DV):
                    w = wpos_v[r, pl.ds(j * _LANES, _LANES)]
                    plsc.addupdate(emb_v.at[r, pl.ds(j * _LANES, _LANES)], w * m)
            pltpu.sync_copy(emb_v, out_hbm.at[pl.ds(base, _SUB)])
            return carry2

        lax.fori_loop(0, _B, b_body, 0)
        return carry

    lax.fori_loop(0, _NSUB, sub_body, 0)


@jax.jit
def _sc_call(emb, sym, wpos):
    mesh = plsc.VectorSubcoreMesh(core_axis_name="c", subcore_axis_name="s")
    fn = pl.kernel(
        _sc_body,
        mesh=mesh,
        out_type=jax.ShapeDtypeStruct((_B * _L, _D), jnp.float32),
        scratch_types=[
            pltpu.VMEM((_SUB, _D), jnp.float32),
            pltpu.VMEM((_SUB, _D), jnp.float32),
            pltpu.VMEM((_SUB,), jnp.int32),
        ],
    )
    return fn(emb, sym, wpos)


def kernel(embedded, symbol, W_pos):
    B, L, D = embedded.shape
    assert (B, L, D) == (_B, _L, _D)
    emb = embedded.reshape(B * L, D)
    sym = symbol.reshape(B * L).astype(jnp.int32)
    out = _sc_call(emb, sym, W_pos[:L])
    return out.reshape(B, L, D)


# speedup vs baseline: 1.6605x; 1.6605x over previous
import jax
import jax.numpy as jnp
from jax import lax
from jax.experimental import pallas as pl
from jax.experimental.pallas import tpu as pltpu
from jax.experimental.pallas import tpu_sc as plsc

_B, _L, _D = 4, 8192, 768
_LANES = 16
_SUB = 32
_NC, _NS = 2, 16
_NW = _NC * _NS
_LW = _L // _NW
_NSUB = _LW // _SUB
_DV = _D // _LANES
_TOT = _NSUB * _B
_ESLOTS = 3
_WSLOTS = 2


def _sc_body(emb_hbm, sym_hbm, wpos_hbm, out_hbm,
             emb_v, wpos_v, sym_v, in_sem, out_sem, wpos_sem, sym_sem):
    c = lax.axis_index("c")
    s = lax.axis_index("s")
    wid = s * _NC + c
    l0w = wid * _LW

    def emb_base(i):
        sub = i // _B
        b = i % _B
        return b * _L + l0w + sub * _SUB

    def issue_in(i, slot):
        pltpu.make_async_copy(
            emb_hbm.at[pl.ds(emb_base(i), _SUB)], emb_v.at[slot],
            in_sem.at[slot]).start()

    def wait_in(slot):
        pltpu.make_async_copy(
            emb_hbm.at[pl.ds(0, _SUB)], emb_v.at[slot],
            in_sem.at[slot]).wait()

    def issue_out(i, slot):
        pltpu.make_async_copy(
            emb_v.at[slot], out_hbm.at[pl.ds(emb_base(i), _SUB)],
            out_sem.at[slot]).start()

    def wait_out(slot):
        pltpu.make_async_copy(
            emb_v.at[slot], out_hbm.at[pl.ds(0, _SUB)],
            out_sem.at[slot]).wait()

    def issue_wpos(sub, slot):
        pltpu.make_async_copy(
            wpos_hbm.at[pl.ds(l0w + sub * _SUB, _SUB)], wpos_v.at[slot],
            wpos_sem.at[slot]).start()

    def wait_wpos(slot):
        pltpu.make_async_copy(
            wpos_hbm.at[pl.ds(0, _SUB)], wpos_v.at[slot],
            wpos_sem.at[slot]).wait()

    for b in range(_B):
        pltpu.make_async_copy(
            sym_hbm.at[pl.ds(b * _L + l0w, _LW)], sym_v.at[b],
            sym_sem).start()
    issue_wpos(0, 0)
    issue_wpos(1, 1)
    issue_in(0, 0)
    issue_in(1, 1)
    for b in range(_B):
        pltpu.make_async_copy(
            sym_hbm.at[pl.ds(0, _LW)], sym_v.at[b], sym_sem).wait()

    def step(i, carry):
        sub = i // _B
        b = i % _B
        eslot = i % _ESLOTS
        wslot = sub % _WSLOTS

        @pl.when(b == 0)
        def _():
            wait_wpos(wslot)

            @pl.when(sub + 1 < _NSUB)
            def _():
                issue_wpos(sub + 1, (sub + 1) % _WSLOTS)

        wait_in(eslot)

        def group(g, carry2):
            row0 = g * _LANES
            svec = sym_v[b, pl.ds(sub * _SUB + row0, _LANES)]
            mvec = jnp.where(svec != 0, jnp.float32(1.0), jnp.float32(0.0))
            for rr in range(_LANES):
                m = mvec[rr]
                r = row0 + rr
                for j in range(_DV):
                    w = wpos_v[wslot, r, pl.ds(j * _LANES, _LANES)]
                    plsc.addupdate(
                        emb_v.at[eslot, r, pl.ds(j * _LANES, _LANES)], w * m)
            return carry2

        lax.fori_loop(0, _SUB // _LANES, group, 0)

        issue_out(i, eslot)

        @pl.when(i + 2 < _TOT)
        def _():
            nslot = (i + 2) % _ESLOTS

            @pl.when(i >= 1)
            def _():
                wait_out(nslot)

            issue_in(i + 2, nslot)

        return carry

    lax.fori_loop(0, _TOT, step, 0)

    for slot in range(_ESLOTS):
        wait_out(slot)


@jax.jit
def _sc_call(emb, sym, wpos):
    mesh = plsc.VectorSubcoreMesh(core_axis_name="c", subcore_axis_name="s")
    fn = pl.kernel(
        _sc_body,
        mesh=mesh,
        out_type=jax.ShapeDtypeStruct((_B * _L, _D), jnp.float32),
        scratch_types=[
            pltpu.VMEM((_ESLOTS, _SUB, _D), jnp.float32),
            pltpu.VMEM((_WSLOTS, _SUB, _D), jnp.float32),
            pltpu.VMEM((_B, _LW), jnp.int32),
            pltpu.SemaphoreType.DMA((_ESLOTS,)),
            pltpu.SemaphoreType.DMA((_ESLOTS,)),
            pltpu.SemaphoreType.DMA((_WSLOTS,)),
            pltpu.SemaphoreType.DMA,
        ],
    )
    return fn(emb, sym, wpos)


def kernel(embedded, symbol, W_pos):
    B, L, D = embedded.shape
    assert (B, L, D) == (_B, _L, _D)
    emb = embedded.reshape(B * L, D)
    sym = symbol.reshape(B * L).astype(jnp.int32)
    out = _sc_call(emb, sym, W_pos[:L])
    return out.reshape(B, L, D)
